# Initial kernel scaffold; baseline (speedup 1.0000x reference)
import jax
import jax.numpy as jnp
from jax.experimental import pallas as pl


def kernel(r_ij, edge_src, w_in, b_in, w_direct, w1, b1, w2, b2, w3, b3, w_a, w_v, w_d):
    raise NotImplementedError("write your pallas kernel here")


# speedup vs baseline: 47.2055x; 47.2055x over previous
import functools

import jax
import jax.numpy as jnp
import numpy as np
from jax import lax
from jax.experimental import pallas as pl
from jax.experimental.pallas import tpu as pltpu
from jax.experimental.pallas import tpu_sc as plsc

N_NODES = 10000
N_EDGES = 160000
L = 2
DIM = 16
DIM_A = 128
DIM_V = 64
DIM_D = 32
R0 = 5.0

PAY_W = 32
ROW_W = 160
CHUNK = 128
NCHUNK = N_EDGES // CHUNK
N_TILES = 16
ROWS_PER_TILE = N_NODES // N_TILES
ZB_ROWS = 25


def _edge_mlp_body(r_ref, w_in_ref, b_in_ref, wd_ref, w1_ref, b1_ref,
                   w2_ref, b2_ref, w3_ref, b3_ref, out_ref):
    r = r_ref[...]
    d2 = jnp.sum(r * r, axis=-1, keepdims=True)
    d = jnp.sqrt(d2 + 1e-12)
    centers = lax.broadcasted_iota(jnp.int32, (1, 8), 1).astype(jnp.float32) * (R0 / 7.0)
    width = R0 / 8.0
    rad = jnp.exp(-((d - centers) ** 2) / (2.0 * width * width))

    h0 = jnp.dot(rad, w_in_ref[...], preferred_element_type=jnp.float32)
    h0 = h0 + b_in_ref[...]
    t = jnp.dot(h0, w1_ref[...], preferred_element_type=jnp.float32) + b1_ref[...]
    t = jnp.where(t >= 0, t, 0.1 * t)
    t = jnp.dot(t, w2_ref[...], preferred_element_type=jnp.float32) + b2_ref[...]
    t = jnp.where(t >= 0, t, 0.1 * t)
    t = jnp.dot(t, w3_ref[...], preferred_element_type=jnp.float32) + b3_ref[...]
    h = jnp.dot(h0, wd_ref[...], preferred_element_type=jnp.float32) + t

    rs = r * (7.0 / R0)
    n = jnp.sqrt(jnp.sum(rs * rs, axis=-1, keepdims=True) + 1e-12)
    rr = rs * (jnp.tanh(n) / n)
    rx = rr[:, 0:1]
    ry = rr[:, 1:2]
    rz = rr[:, 2:3]
    ones = jnp.ones_like(rx)
    zeros6 = jnp.zeros((rx.shape[0], 6), jnp.float32)
    m = jnp.concatenate(
        [ones, rx, ry, rz,
         rx * rx, rx * ry, rx * rz, ry * ry, ry * rz, rz * rz, zeros6],
        axis=1)
    out_ref[...] = jnp.concatenate([h, m], axis=1)


def _edge_mlp(r_flat, w_in, b_in, w_direct, w1, b1, w2, b2, w3, b3):
    eb = 3200
    grid = (r_flat.shape[0] // eb,)
    full = lambda shape: pl.BlockSpec(shape, lambda i: (0, 0))
    return pl.pallas_call(
        _edge_mlp_body,
        grid=grid,
        in_specs=[
            pl.BlockSpec((eb, 3), lambda i: (i, 0)),
            full((8, DIM)), full((1, DIM)), full((DIM, DIM)),
            full((DIM, 2 * DIM)), full((1, 2 * DIM)),
            full((2 * DIM, 2 * DIM)), full((1, 2 * DIM)),
            full((2 * DIM, DIM)), full((1, DIM)),
        ],
        out_specs=pl.BlockSpec((eb, PAY_W), lambda i: (i, 0)),
        out_shape=jax.ShapeDtypeStruct((r_flat.shape[0], PAY_W), jnp.float32),
    )(r_flat, w_in, b_in.reshape(1, -1), w_direct,
      w1, b1.reshape(1, -1), w2, b2.reshape(1, -1), w3, b3.reshape(1, -1))


def _sc_scatter_body(pay_hbm, src_hbm, out_hbm, idx_v, pay_v, row_v, zb_v, table):
    c = lax.axis_index("c")
    s = lax.axis_index("s")

    zv = jnp.zeros((16,), jnp.float32)

    def zrow(i, carry):
        for k in range(ROW_W // 16):
            zb_v[i, pl.ds(k * 16, 16)] = zv
        return carry

    lax.fori_loop(0, ZB_ROWS, zrow, 0)
    for b in range(ROWS_PER_TILE // ZB_ROWS):
        pltpu.sync_copy(zb_v, table.at[pl.ds(s * ROWS_PER_TILE + b * ZB_ROWS, ZB_ROWS)])
    plsc.subcore_barrier()

    n_iters = (NCHUNK - s + N_TILES - 1) // N_TILES

    def chunk_body(i, carry):
        j = s + i * N_TILES
        base = c * N_EDGES + j * CHUNK
        pltpu.sync_copy(pay_hbm.at[pl.ds(base, CHUNK)], pay_v)
        pltpu.sync_copy(src_hbm.at[pl.ds(j * CHUNK, CHUNK)], idx_v.at[0])

        def edge(e, ecarry):
            h = pay_v[e, pl.ds(0, 16)]
            mvec = pay_v[e, pl.ds(16, 16)]
            for k in range(10):
                row_v[e, pl.ds(k * 16, 16)] = h * mvec[k]
            return ecarry

        lax.fori_loop(0, CHUNK, edge, 0)
        pltpu.sync_copy(row_v, table.at[idx_v.at[0]], add=True)
        return carry

    lax.fori_loop(0, n_iters, chunk_body, 0)
    plsc.subcore_barrier()

    pltpu.sync_copy(table.at[pl.ds(s * ROWS_PER_TILE, ROWS_PER_TILE)],
                    out_hbm.at[c, pl.ds(s * ROWS_PER_TILE, ROWS_PER_TILE)])


def _sc_scatter(payload, edge_src):
    mesh = plsc.VectorSubcoreMesh(core_axis_name="c", subcore_axis_name="s")
    f = pl.kernel(
        _sc_scatter_body,
        out_type=jax.ShapeDtypeStruct((L, N_NODES, ROW_W), jnp.float32),
        mesh=mesh,
        scratch_types=[
            pltpu.VMEM((1, CHUNK), jnp.int32),
            pltpu.VMEM((CHUNK, PAY_W), jnp.float32),
            pltpu.VMEM((CHUNK, ROW_W), jnp.float32),
            pltpu.VMEM((ZB_ROWS, ROW_W), jnp.float32),
            pltpu.VMEM_SHARED((N_NODES, ROW_W), jnp.float32),
        ],
        compiler_params=pltpu.CompilerParams(use_tc_tiling_on_sc=False),
    )
    return f(payload, edge_src)


def _node_mix_body(t_ref, wa_ref, wv_ref, wd_ref, oa_ref, ov_ref, od_ref):
    t = t_ref[...]

    def fused(k):
        return jnp.concatenate([t[0, :, k * 16:(k + 1) * 16],
                                t[1, :, k * 16:(k + 1) * 16]], axis=1)

    wa = wa_ref[...]
    wv = wv_ref[...]
    wd = wd_ref[...]
    oa_ref[...] = jnp.dot(fused(0), wa, preferred_element_type=jnp.float32)
    for x in range(3):
        ov_ref[x, :, :] = jnp.dot(fused(1 + x), wv,
                                  preferred_element_type=jnp.float32)
    for q in range(6):
        od_ref[q, :, :] = jnp.dot(fused(4 + q), wd,
                                  preferred_element_type=jnp.float32)


def _node_mix(tables, w_a, w_v, w_d):
    nb = 2000
    grid = (N_NODES // nb,)
    full = lambda shape: pl.BlockSpec(shape, lambda i: tuple(0 for _ in shape))
    return pl.pallas_call(
        _node_mix_body,
        grid=grid,
        in_specs=[
            pl.BlockSpec((L, nb, ROW_W), lambda i: (0, i, 0)),
            full((L * DIM, DIM_A)), full((L * DIM, DIM_V)), full((L * DIM, DIM_D)),
        ],
        out_specs=[
            pl.BlockSpec((nb, DIM_A), lambda i: (i, 0)),
            pl.BlockSpec((3, nb, DIM_V), lambda i: (0, i, 0)),
            pl.BlockSpec((6, nb, DIM_D), lambda i: (0, i, 0)),
        ],
        out_shape=[
            jax.ShapeDtypeStruct((N_NODES, DIM_A), jnp.float32),
            jax.ShapeDtypeStruct((3, N_NODES, DIM_V), jnp.float32),
            jax.ShapeDtypeStruct((6, N_NODES, DIM_D), jnp.float32),
        ],
    )(tables, w_a, w_v, w_d)


def kernel(r_ij, edge_src, w_in, b_in, w_direct, w1, b1, w2, b2, w3, b3,
           w_a, w_v, w_d):
    r_flat = r_ij.reshape(L * N_EDGES, 3)
    payload = _edge_mlp(r_flat, w_in, b_in, w_direct, w1, b1, w2, b2, w3, b3)
    tables = _sc_scatter(payload, edge_src)
    out_a, out_v, out_d = _node_mix(tables, w_a, w_v, w_d)

    out_a = out_a[None]
    out_v = jnp.transpose(out_v, (1, 2, 0))[None]
    sym = jnp.array([[0, 1, 2], [1, 3, 4], [2, 4, 5]], jnp.int32)
    out_d = jnp.transpose(out_d[sym], (2, 3, 0, 1))[None]
    return (out_a, out_v, out_d)
